# baseline (speedup 1.0000x reference)
import functools

import jax
import jax.numpy as jnp
from jax import lax
from jax.experimental import pallas as pl
from jax.experimental.pallas import tpu as pltpu
from jax.experimental.pallas import tpu_sc as plsc

_NC = 2
_NS = 16
_NW = _NC * _NS

_CHUNK = 128
_NBUF = 8
_DEPTH = 4


def _sc_gather(embed, atn_chunked, dtype=jnp.float32):
    n_chunks_total, chunk = atn_chunked.shape
    H = embed.shape[1]
    R = n_chunks_total * chunk
    cpw = n_chunks_total // _NW

    mesh = plsc.VectorSubcoreMesh(core_axis_name="c", subcore_axis_name="s",
                                  num_cores=_NC, num_subcores=_NS)

    @functools.partial(
        pl.kernel,
        mesh=mesh,
        out_type=jax.ShapeDtypeStruct((R, H), dtype),
        scratch_types=(
            [pltpu.VMEM((cpw, chunk), jnp.int32)]
            + [pltpu.VMEM((chunk, H), dtype) for _ in range(_NBUF)]
            + [pltpu.SemaphoreType.DMA for _ in range(2 * _NBUF)]
        ),
        compiler_params=pltpu.CompilerParams(needs_layout_passes=False,
                                             use_tc_tiling_on_sc=False),
    )
    def gather_kernel(embed_hbm, idx_hbm, out_hbm, idx_v, *rest):
        bufs = rest[:_NBUF]
        gsem = rest[_NBUF:2 * _NBUF]
        ssem = rest[2 * _NBUF:]
        wid = lax.axis_index("s") * _NC + lax.axis_index("c")
        cbase = wid * cpw
        rbase = cbase * chunk
        pltpu.sync_copy(idx_hbm.at[pl.ds(cbase, cpw)], idx_v)

        gd, sd = {}, {}

        def start_gather(c):
            j = c % _NBUF
            gd[c] = pltpu.async_copy(embed_hbm.at[idx_v.at[c]], bufs[j],
                                     gsem[j])

        def start_store(c):
            j = c % _NBUF
            sd[c] = pltpu.async_copy(
                bufs[j], out_hbm.at[pl.ds(rbase + c * chunk, chunk)], ssem[j])

        for c in range(min(_DEPTH, cpw)):
            start_gather(c)
        for s in range(cpw):
            gd.pop(s).wait()
            start_store(s)
            p = s + _DEPTH
            if p < cpw:
                if p >= _NBUF:
                    sd.pop(p - _NBUF).wait()
                start_gather(p)
        for c in sorted(sd):
            sd.pop(c).wait()

    return gather_kernel(embed, atn_chunked)


def _tc_score(targs3, stim, W1, b1, Wfc, bfc):
    B, N, H = targs3.shape
    BLK = 256
    b1r = b1.reshape(1, 1, H)
    wfcr = Wfc.reshape(1, 1, H)
    bfcr = bfc.reshape(1, 1)

    def body(t_ref, stim_ref, w1_ref, b1_ref, wfc_ref, bfc_ref,
             attn_ref, idx_ref):
        t2 = t_ref[...].reshape(BLK * N, H)
        if t2.dtype != jnp.bfloat16:
            t2 = t2.astype(jnp.bfloat16)
        x2 = lax.dot_general(
            t2, w1_ref[...].astype(jnp.bfloat16),
            (((1,), (0,)), ((), ())), preferred_element_type=jnp.float32)
        x3 = x2.reshape(BLK, N, H) + b1_ref[...]
        y3 = x3 * stim_ref[...].reshape(BLK, 1, H)
        wfc_b = wfc_ref[...].astype(jnp.bfloat16).astype(jnp.float32)
        z3 = y3.astype(jnp.bfloat16).astype(jnp.float32) * wfc_b
        attn = jnp.sum(z3, axis=2) + bfc_ref[...]
        attn_ref[...] = attn
        m = jnp.max(attn, axis=1, keepdims=True)
        n_iota = lax.broadcasted_iota(jnp.int32, (BLK, N), 1)
        cand = jnp.where(attn == m, n_iota, N)
        idx_ref[...] = jnp.min(cand, axis=1, keepdims=True)

    grid = (B // BLK,)
    attn, idx2 = pl.pallas_call(
        body,
        grid=grid,
        in_specs=[
            pl.BlockSpec((BLK, N, H), lambda i: (i, 0, 0)),
            pl.BlockSpec((BLK, H), lambda i: (i, 0)),
            pl.BlockSpec((H, H), lambda i: (0, 0)),
            pl.BlockSpec((1, 1, H), lambda i: (0, 0, 0)),
            pl.BlockSpec((1, 1, H), lambda i: (0, 0, 0)),
            pl.BlockSpec((1, 1), lambda i: (0, 0)),
        ],
        out_specs=[
            pl.BlockSpec((BLK, N), lambda i: (i, 0)),
            pl.BlockSpec((BLK, 1), lambda i: (i, 0)),
        ],
        out_shape=[
            jax.ShapeDtypeStruct((B, N), jnp.float32),
            jax.ShapeDtypeStruct((B, 1), jnp.int32),
        ],
    )(targs3, stim, W1, b1r, wfcr, bfcr)
    return attn, idx2


def kernel(stim, atnTensor, embed, W1, b1, Wfc, bfc):
    B, N = atnTensor.shape
    V, H = embed.shape
    atn_chunked = atnTensor.reshape(B * N // _CHUNK, _CHUNK)
    emb_bf = embed.astype(jnp.bfloat16)
    targs_bf = _sc_gather(emb_bf, atn_chunked, dtype=jnp.bfloat16)
    attn, idx2 = _tc_score(targs_bf.reshape(B, N, H), stim, W1, b1, Wfc, bfc)
    return attn, idx2.reshape(B)


# speedup vs baseline: 1.2737x; 1.2737x over previous
import functools

import jax
import jax.numpy as jnp
from jax import lax
from jax.experimental import pallas as pl
from jax.experimental.pallas import tpu as pltpu
from jax.experimental.pallas import tpu_sc as plsc

_NC = 2
_NS = 16
_NW = _NC * _NS

_CHUNK = 128
_NBUF = 8
_DEPTH = 4


def _sc_gather(embed, atn_chunked, dtype=jnp.float32):
    n_chunks_total, chunk = atn_chunked.shape
    H = embed.shape[1]
    R = n_chunks_total * chunk
    cpw = n_chunks_total // _NW

    mesh = plsc.VectorSubcoreMesh(core_axis_name="c", subcore_axis_name="s",
                                  num_cores=_NC, num_subcores=_NS)

    @functools.partial(
        pl.kernel,
        mesh=mesh,
        out_type=jax.ShapeDtypeStruct((R, H), dtype),
        scratch_types=(
            [pltpu.VMEM((cpw, chunk), jnp.int32)]
            + [pltpu.VMEM((chunk, H), dtype) for _ in range(_NBUF)]
            + [pltpu.SemaphoreType.DMA for _ in range(2 * _NBUF)]
        ),
        compiler_params=pltpu.CompilerParams(needs_layout_passes=False,
                                             use_tc_tiling_on_sc=False),
    )
    def gather_kernel(embed_hbm, idx_hbm, out_hbm, idx_v, *rest):
        bufs = rest[:_NBUF]
        gsem = rest[_NBUF:2 * _NBUF]
        ssem = rest[2 * _NBUF:]
        wid = lax.axis_index("s") * _NC + lax.axis_index("c")
        cbase = wid * cpw
        rbase = cbase * chunk
        pltpu.sync_copy(idx_hbm.at[pl.ds(cbase, cpw)], idx_v)

        gd, sd = {}, {}

        def start_gather(c):
            j = c % _NBUF
            gd[c] = pltpu.async_copy(embed_hbm.at[idx_v.at[c]], bufs[j],
                                     gsem[j])

        def start_store(c):
            j = c % _NBUF
            sd[c] = pltpu.async_copy(
                bufs[j], out_hbm.at[pl.ds(rbase + c * chunk, chunk)], ssem[j])

        for c in range(min(_DEPTH, cpw)):
            start_gather(c)
        for s in range(cpw):
            gd.pop(s).wait()
            start_store(s)
            p = s + _DEPTH
            if p < cpw:
                if p >= _NBUF:
                    sd.pop(p - _NBUF).wait()
                start_gather(p)
        for c in sorted(sd):
            sd.pop(c).wait()

    return gather_kernel(embed, atn_chunked)


def _tc_score(targs_p, stim, W1, b1, Wfc, bfc):
    B, Nh, H2 = targs_p.shape
    N = Nh * 2
    H = H2 // 2
    BLK = 256
    bfcr = bfc.reshape(1, 1)

    zeros = jnp.zeros_like(W1)
    w1bd = jnp.concatenate(
        [jnp.concatenate([W1, zeros], axis=0),
         jnp.concatenate([zeros, W1], axis=0)], axis=1)
    b1p = jnp.concatenate([b1, b1]).reshape(1, 1, H2)
    wfc_flat = Wfc.reshape(H)
    zf = jnp.zeros_like(wfc_flat)
    wfce = jnp.concatenate([wfc_flat, zf]).reshape(1, 1, H2)
    wfco = jnp.concatenate([zf, wfc_flat]).reshape(1, 1, H2)
    stimp = jnp.concatenate([stim, stim], axis=1)

    def body(t_ref, stimp_ref, w1bd_ref, b1p_ref, wfce_ref, wfco_ref,
             bfc_ref, attn_e_ref, attn_o_ref, idx_ref):
        t2 = t_ref[...].reshape(BLK * Nh, H2).astype(jnp.bfloat16)
        x2 = lax.dot_general(t2, w1bd_ref[...].astype(jnp.bfloat16),
                             (((1,), (0,)), ((), ())),
                             preferred_element_type=jnp.float32)
        x3 = x2.reshape(BLK, Nh, H2) + b1p_ref[...]
        y3 = x3 * stimp_ref[...].reshape(BLK, 1, H2)
        z3 = y3.astype(jnp.bfloat16).astype(jnp.float32)
        wfce_b = wfce_ref[...].astype(jnp.bfloat16).astype(jnp.float32)
        wfco_b = wfco_ref[...].astype(jnp.bfloat16).astype(jnp.float32)
        attn_e = jnp.sum(z3 * wfce_b, axis=2) + bfc_ref[...]
        attn_o = jnp.sum(z3 * wfco_b, axis=2) + bfc_ref[...]
        attn_e_ref[...] = attn_e
        attn_o_ref[...] = attn_o
        m = jnp.maximum(jnp.max(attn_e, axis=1, keepdims=True),
                        jnp.max(attn_o, axis=1, keepdims=True))
        n_iota = lax.broadcasted_iota(jnp.int32, (BLK, Nh), 1)
        cand_e = jnp.where(attn_e == m, 2 * n_iota, N)
        cand_o = jnp.where(attn_o == m, 2 * n_iota + 1, N)
        idx_ref[...] = jnp.minimum(
            jnp.min(cand_e, axis=1, keepdims=True),
            jnp.min(cand_o, axis=1, keepdims=True))

    grid = (B // BLK,)
    attn_e, attn_o, idx2 = pl.pallas_call(
        body,
        grid=grid,
        in_specs=[
            pl.BlockSpec((BLK, Nh, H2), lambda i: (i, 0, 0)),
            pl.BlockSpec((BLK, H2), lambda i: (i, 0)),
            pl.BlockSpec((H2, H2), lambda i: (0, 0)),
            pl.BlockSpec((1, 1, H2), lambda i: (0, 0, 0)),
            pl.BlockSpec((1, 1, H2), lambda i: (0, 0, 0)),
            pl.BlockSpec((1, 1, H2), lambda i: (0, 0, 0)),
            pl.BlockSpec((1, 1), lambda i: (0, 0)),
        ],
        out_specs=[
            pl.BlockSpec((BLK, Nh), lambda i: (i, 0)),
            pl.BlockSpec((BLK, Nh), lambda i: (i, 0)),
            pl.BlockSpec((BLK, 1), lambda i: (i, 0)),
        ],
        out_shape=[
            jax.ShapeDtypeStruct((B, Nh), jnp.float32),
            jax.ShapeDtypeStruct((B, Nh), jnp.float32),
            jax.ShapeDtypeStruct((B, 1), jnp.int32),
        ],
    )(targs_p, stimp, w1bd, b1p, wfce, wfco, bfcr)
    return attn_e, attn_o, idx2


def kernel(stim, atnTensor, embed, W1, b1, Wfc, bfc):
    B, N = atnTensor.shape
    V, H = embed.shape
    atn_chunked = atnTensor.reshape(B * N // _CHUNK, _CHUNK)
    targs = _sc_gather(embed, atn_chunked)
    targs_p = targs.reshape(B, N // 2, 2 * H)
    attn_e, attn_o, idx2 = _tc_score(targs_p, stim, W1, b1, Wfc, bfc)
    attn = jnp.stack([attn_e, attn_o], axis=-1).reshape(B, N)
    return attn, idx2.reshape(B)


# speedup vs baseline: 1.6631x; 1.3057x over previous
import functools

import jax
import jax.numpy as jnp
from jax import lax
from jax.experimental import pallas as pl
from jax.experimental.pallas import tpu as pltpu
from jax.experimental.pallas import tpu_sc as plsc

_NC = 2
_NS = 16
_NW = _NC * _NS

_CHUNK = 128
_NBUF = 6
_DEPTH = 3


def _sc_gather(embed_p, atn_chunked, dtype=jnp.float32):
    n_chunks_total, chunk = atn_chunked.shape
    Vh, H2 = embed_p.shape
    R = n_chunks_total * chunk
    cpw = n_chunks_total // _NW

    mesh = plsc.VectorSubcoreMesh(core_axis_name="c", subcore_axis_name="s",
                                  num_cores=_NC, num_subcores=_NS)

    @functools.partial(
        pl.kernel,
        mesh=mesh,
        out_type=jax.ShapeDtypeStruct((R, H2), dtype),
        scratch_types=(
            [pltpu.VMEM((cpw, chunk), jnp.int32)]
            + [pltpu.VMEM((chunk, H2), dtype) for _ in range(_NBUF)]
            + [pltpu.SemaphoreType.DMA for _ in range(2 * _NBUF)]
        ),
        compiler_params=pltpu.CompilerParams(needs_layout_passes=False,
                                             use_tc_tiling_on_sc=False),
    )
    def gather_kernel(embed_hbm, idx_hbm, out_hbm, idx_v, *rest):
        bufs = rest[:_NBUF]
        gsem = rest[_NBUF:2 * _NBUF]
        ssem = rest[2 * _NBUF:]
        wid = lax.axis_index("s") * _NC + lax.axis_index("c")
        cbase = wid * cpw
        rbase = cbase * chunk
        pltpu.sync_copy(idx_hbm.at[pl.ds(cbase, cpw)], idx_v)

        gd, sd = {}, {}

        def start_gather(c):
            j = c % _NBUF
            gd[c] = pltpu.async_copy(embed_hbm.at[idx_v.at[c]], bufs[j],
                                     gsem[j])

        def start_store(c):
            j = c % _NBUF
            sd[c] = pltpu.async_copy(
                bufs[j], out_hbm.at[pl.ds(rbase + c * chunk, chunk)], ssem[j])

        for c in range(min(_DEPTH, cpw)):
            start_gather(c)
        for s in range(cpw):
            gd.pop(s).wait()
            start_store(s)
            p = s + _DEPTH
            if p < cpw:
                if p >= _NBUF:
                    sd.pop(p - _NBUF).wait()
                start_gather(p)
        for c in sorted(sd):
            sd.pop(c).wait()

    return gather_kernel(embed_p, atn_chunked)


def _tc_transpose(embT, I64):
    H, V = embT.shape
    C = 2048
    Ch = C // 2
    grid = ((V + C - 1) // C,)
    Vp = grid[0] * Ch

    def body(t_ref, i64_ref, out_ref):
        ident = i64_ref[...]
        t = t_ref[...]
        t1 = t[:, :Ch].astype(jnp.bfloat16).astype(jnp.float32)
        t2 = t[:, Ch:].astype(jnp.bfloat16).astype(jnp.float32)
        tr1 = lax.dot_general(t1, ident, (((0,), (0,)), ((), ())),
                              preferred_element_type=jnp.float32)
        tr2 = lax.dot_general(t2, ident, (((0,), (0,)), ((), ())),
                              preferred_element_type=jnp.float32)
        out_ref[...] = jnp.concatenate([tr1, tr2], axis=1)

    return pl.pallas_call(
        body,
        grid=grid,
        in_specs=[
            pl.BlockSpec((H, C), lambda i: (0, i)),
            pl.BlockSpec((H, H), lambda i: (0, 0)),
        ],
        out_specs=pl.BlockSpec((Ch, 2 * H), lambda i: (i, 0)),
        out_shape=jax.ShapeDtypeStruct((Vp, 2 * H), jnp.float32),
    )(embT, I64)


def _tc_score(targs_p, par, stim, W1, b1, Wfc, bfc):
    Rp, H2 = targs_p.shape
    B, N = par.shape
    H = H2 // 2
    BLK = 128
    bfcr = bfc.reshape(1, 1)

    zeros = jnp.zeros_like(W1)
    w1bd = jnp.concatenate(
        [jnp.concatenate([W1, zeros], axis=0),
         jnp.concatenate([zeros, W1], axis=0)], axis=1)
    b1p = jnp.concatenate([b1, b1]).reshape(1, 1, H2)
    wfc_flat = Wfc.reshape(H)
    zf = jnp.zeros_like(wfc_flat)
    wfce = jnp.concatenate([wfc_flat, zf]).reshape(1, 1, H2)
    wfco = jnp.concatenate([zf, wfc_flat]).reshape(1, 1, H2)
    stimp = jnp.concatenate([stim, stim], axis=1)

    def body(t_ref, par_ref, stimp_ref, w1bd_ref, b1p_ref, wfce_ref,
             wfco_ref, bfc_ref, attn_ref, idx_ref):
        t2 = t_ref[...].astype(jnp.bfloat16)
        x2 = lax.dot_general(t2, w1bd_ref[...].astype(jnp.bfloat16),
                             (((1,), (0,)), ((), ())),
                             preferred_element_type=jnp.float32)
        x3 = x2.reshape(BLK, N, H2) + b1p_ref[...]
        y3 = x3 * stimp_ref[...].reshape(BLK, 1, H2)
        z3 = y3.astype(jnp.bfloat16).astype(jnp.float32)
        wfce_b = wfce_ref[...].astype(jnp.bfloat16).astype(jnp.float32)
        wfco_b = wfco_ref[...].astype(jnp.bfloat16).astype(jnp.float32)
        par3 = par_ref[...].reshape(BLK, N, 1)
        mask = jnp.where(par3 == 1, wfco_b, wfce_b)
        attn = jnp.sum(z3 * mask, axis=2) + bfc_ref[...]
        attn_ref[...] = attn
        m = jnp.max(attn, axis=1, keepdims=True)
        n_iota = lax.broadcasted_iota(jnp.int32, (BLK, N), 1)
        cand = jnp.where(attn == m, n_iota, N)
        idx_ref[...] = jnp.min(cand, axis=1, keepdims=True)

    grid = (B // BLK,)
    attn, idx2 = pl.pallas_call(
        body,
        grid=grid,
        in_specs=[
            pl.BlockSpec((BLK * N, H2), lambda i: (i, 0)),
            pl.BlockSpec((BLK, N), lambda i: (i, 0)),
            pl.BlockSpec((BLK, H2), lambda i: (i, 0)),
            pl.BlockSpec((H2, H2), lambda i: (0, 0)),
            pl.BlockSpec((1, 1, H2), lambda i: (0, 0, 0)),
            pl.BlockSpec((1, 1, H2), lambda i: (0, 0, 0)),
            pl.BlockSpec((1, 1, H2), lambda i: (0, 0, 0)),
            pl.BlockSpec((1, 1), lambda i: (0, 0)),
        ],
        out_specs=[
            pl.BlockSpec((BLK, N), lambda i: (i, 0)),
            pl.BlockSpec((BLK, 1), lambda i: (i, 0)),
        ],
        out_shape=[
            jax.ShapeDtypeStruct((B, N), jnp.float32),
            jax.ShapeDtypeStruct((B, 1), jnp.int32),
        ],
        compiler_params=pltpu.CompilerParams(
            vmem_limit_bytes=100 * 1024 * 1024),
    )(targs_p, par, stimp, w1bd, b1p, wfce, wfco, bfcr)
    return attn, idx2


def kernel(stim, atnTensor, embed, W1, b1, Wfc, bfc):
    B, N = atnTensor.shape
    V, H = embed.shape
    blk = atnTensor // 2048
    q = atnTensor % 2048
    atn_pair = (blk * 1024 + q % 1024).reshape(B * N // _CHUNK, _CHUNK)
    par = q // 1024
    I64 = jnp.eye(H, dtype=jnp.float32)
    emb_packed = _tc_transpose(embed.T, I64)
    targs_p = _sc_gather(emb_packed, atn_pair)
    attn, idx2 = _tc_score(targs_p, par, stim, W1, b1, Wfc, bfc)
    return attn, idx2.reshape(B)
